# Initial kernel scaffold; baseline (speedup 1.0000x reference)
import jax
import jax.numpy as jnp
from jax.experimental import pallas as pl


def kernel(x, edge_index, edge_weight, W, b):
    raise NotImplementedError("write your pallas kernel here")


# speedup vs baseline: 3.4311x; 3.4311x over previous
import functools

import jax
import jax.numpy as jnp
from jax import lax
from jax.experimental import pallas as pl
from jax.experimental.pallas import tpu as pltpu
from jax.experimental.pallas import tpu_sc as plsc

N = 10000
NP = 10240
D = 256
DH = 128
NC = 2
NS = 16
CH = 128
ROWS_PER_SUB = NP // NS
ZROWS = 128

_mesh = plsc.VectorSubcoreMesh(
    core_axis_name="c", subcore_axis_name="s", num_cores=NC, num_subcores=NS)


@functools.lru_cache(maxsize=None)
def _make_spmm(nchunks):
    @functools.partial(
        pl.kernel,
        out_type=(jax.ShapeDtypeStruct((NP, DH), jnp.float32),
                  jax.ShapeDtypeStruct((NP, DH), jnp.float32)),
        mesh=_mesh,
        scratch_types=[
            pltpu.VMEM((nchunks, CH), jnp.int32),
            pltpu.VMEM((nchunks, CH), jnp.int32),
            pltpu.VMEM((nchunks * CH,), jnp.float32),
            pltpu.VMEM((CH, DH), jnp.float32),
            pltpu.VMEM_SHARED((NP, DH), jnp.float32),
            pltpu.SemaphoreType.DMA,
        ],
    )
    def spmm(src_lo, src_hi, rows3, cols3, w2, out_lo, out_hi,
             rows_all, cols_all, w_all, gbuf, acc, gsem):
        c = lax.axis_index("c")
        s = lax.axis_index("s")

        pltpu.sync_copy(rows3.at[s], rows_all)
        pltpu.sync_copy(cols3.at[s], cols_all)
        pltpu.sync_copy(w2.at[s], w_all)

        def zrow(r, carry):
            for jj in range(DH // 16):
                gbuf[r, pl.ds(jj * 16, 16)] = jnp.zeros((16,), jnp.float32)
            return carry
        lax.fori_loop(0, ZROWS, zrow, 0)
        for k in range(ROWS_PER_SUB // ZROWS):
            pltpu.sync_copy(
                gbuf, acc.at[pl.ds(s * ROWS_PER_SUB + k * ZROWS, ZROWS)])
        plsc.subcore_barrier()

        def run(src_hbm, out_hbm):
            def chunk(j, carry):
                pltpu.async_copy(src_hbm.at[cols_all.at[j]], gbuf, gsem).wait()

                def scale(g, gcarry):
                    wv16 = w_all[pl.ds(j * CH + g * 16, 16)]
                    for l in range(16):
                        wb = lax.gather(
                            wv16, jnp.full((16, 1), l, jnp.int32),
                            lax.GatherDimensionNumbers(
                                offset_dims=(), collapsed_slice_dims=(0,),
                                start_index_map=(0,)),
                            (1,), mode=lax.GatherScatterMode.PROMISE_IN_BOUNDS)
                        i = g * 16 + l
                        for jj in range(DH // 16):
                            sl = pl.ds(jj * 16, 16)
                            gbuf[i, sl] = gbuf[i, sl] * wb
                    return gcarry
                lax.fori_loop(0, CH // 16, scale, 0)
                pltpu.sync_copy(gbuf, acc.at[rows_all.at[j]], add=True)
                return carry
            lax.fori_loop(0, nchunks, chunk, 0)
            plsc.subcore_barrier()
            base = s * ROWS_PER_SUB
            pltpu.sync_copy(acc.at[pl.ds(base, ROWS_PER_SUB)],
                            out_hbm.at[pl.ds(base, ROWS_PER_SUB)])

        @pl.when(c == 0)
        def _():
            run(src_lo, out_lo)

        @pl.when(c == 1)
        def _():
            run(src_hi, out_hi)

    return spmm


BN = 400


def _dense_body(x_b, h1lo_b, h1hi_b, h2lo_b, h2hi_b,
                wx, w1lo, w1hi, w2lo, w2hi, b_b, out_b):
    acc = jnp.dot(x_b[...], wx[...], preferred_element_type=jnp.float32)
    acc += jnp.dot(h1lo_b[...], w1lo[...], preferred_element_type=jnp.float32)
    acc += jnp.dot(h1hi_b[...], w1hi[...], preferred_element_type=jnp.float32)
    acc += jnp.dot(h2lo_b[...], w2lo[...], preferred_element_type=jnp.float32)
    acc += jnp.dot(h2hi_b[...], w2hi[...], preferred_element_type=jnp.float32)
    out_b[...] = acc + b_b[...]


_dense = pl.pallas_call(
    _dense_body,
    grid=(N // BN,),
    in_specs=[
        pl.BlockSpec((BN, D), lambda i: (i, 0)),
        pl.BlockSpec((BN, DH), lambda i: (i, 0)),
        pl.BlockSpec((BN, DH), lambda i: (i, 0)),
        pl.BlockSpec((BN, DH), lambda i: (i, 0)),
        pl.BlockSpec((BN, DH), lambda i: (i, 0)),
        pl.BlockSpec((D, D), lambda i: (0, 0)),
        pl.BlockSpec((DH, D), lambda i: (0, 0)),
        pl.BlockSpec((DH, D), lambda i: (0, 0)),
        pl.BlockSpec((DH, D), lambda i: (0, 0)),
        pl.BlockSpec((DH, D), lambda i: (0, 0)),
        pl.BlockSpec((1, D), lambda i: (0, 0)),
    ],
    out_specs=pl.BlockSpec((BN, D), lambda i: (i, 0)),
    out_shape=jax.ShapeDtypeStruct((N, D), jnp.float32),
)


def kernel(x, edge_index, edge_weight, W, b):
    e = edge_index.shape[1]
    nchunks = -(-e // (NS * CH))
    ep = NS * CH * nchunks
    rows = jnp.pad(edge_index[0], (0, ep - e))
    cols = jnp.pad(edge_index[1], (0, ep - e))
    w = jnp.pad(edge_weight, (0, ep - e))
    rows3 = rows.reshape(NS, nchunks, CH)
    cols3 = cols.reshape(NS, nchunks, CH)
    w2 = w.reshape(NS, nchunks * CH)

    x_lo = x[:, :DH]
    x_hi = x[:, DH:]
    spmm = _make_spmm(nchunks)
    h1_lo, h1_hi = spmm(x_lo, x_hi, rows3, cols3, w2)
    h2_lo, h2_hi = spmm(h1_lo, h1_hi, rows3, cols3, w2)

    wt = W.T
    out = _dense(x, h1_lo[:N], h1_hi[:N], h2_lo[:N], h2_hi[:N],
                 wt[:D], wt[D:D + DH], wt[D + DH:2 * D],
                 wt[2 * D:2 * D + DH], wt[2 * D + DH:],
                 b.reshape(1, D))
    return out
